# Initial kernel scaffold; baseline (speedup 1.0000x reference)
import jax
import jax.numpy as jnp
from jax.experimental import pallas as pl


def kernel(inputs, label, weight):
    raise NotImplementedError("write your pallas kernel here")


# speedup vs baseline: 4.8067x; 4.8067x over previous
import jax
import jax.numpy as jnp
from jax.experimental import pallas as pl
from jax.experimental.pallas import tpu as pltpu

NUM_EMBEDDINGS = 1024
EMBEDDING_DIM = 64
BATCH = 16384
COMMITMENT_COST = 0.25
DIVERGENCE_COST = 0.1

_BB = 512
_NB = BATCH // _BB


def _vq_body(x_ref, lab_ref, w_ref, loss_ref, qst_ref, perp_ref, enc_ref,
             sums_ref, counts_ref):
    i = pl.program_id(0)
    x = x_ref[...]
    w = w_ref[...]
    lab = lab_ref[0, 0, :]

    x2 = jnp.sum(x * x, axis=1, keepdims=True)
    w2 = jnp.sum(w * w, axis=1)
    xwt = jax.lax.dot_general(x, w, (((1,), (1,)), ((), ())),
                              preferred_element_type=jnp.float32)
    dist = x2 + w2[None, :] - 2.0 * xwt

    colids = jax.lax.broadcasted_iota(jnp.int32, (_BB, NUM_EMBEDDINGS), 1)
    dmin = jnp.min(dist, axis=1, keepdims=True)
    close = jnp.min(jnp.where(dist <= dmin, colids, NUM_EMBEDDINGS),
                    axis=1)

    enc = (colids == lab[:, None]).astype(jnp.float32)
    cenc = (colids == close[:, None]).astype(jnp.float32)

    q = jax.lax.dot_general(enc, w, (((1,), (0,)), ((), ())),
                            preferred_element_type=jnp.float32)
    cq = jax.lax.dot_general(cenc, w, (((1,), (0,)), ((), ())),
                             preferred_element_type=jnp.float32)

    ind = (lab != close).astype(jnp.float32)
    dq = q - x
    dcq = cq - x
    s1 = jnp.sum(dq * dq)
    s2 = jnp.sum(ind[:, None] * (dcq * dcq))

    enc_ref[...] = enc
    qst_ref[...] = x + dq

    @pl.when(i == 0)
    def _init():
        sums_ref[0] = s1
        sums_ref[1] = s2
        counts_ref[...] = jnp.sum(enc, axis=0)

    @pl.when(i > 0)
    def _acc():
        sums_ref[0] += s1
        sums_ref[1] += s2
        counts_ref[...] += jnp.sum(enc, axis=0)

    @pl.when(i == _NB - 1)
    def _finish():
        denom = float(BATCH * EMBEDDING_DIM)
        m1 = sums_ref[0] / denom
        m2 = sums_ref[1] / denom
        loss_ref[...] = jnp.reshape((1.0 + COMMITMENT_COST) * m1
                                    - (1.0 + DIVERGENCE_COST) * m2, (1, 1))
        p = counts_ref[...] / float(BATCH)
        perp_ref[...] = jnp.reshape(jnp.exp(-jnp.sum(p * jnp.log(p + 1e-10))),
                                    (1, 1))


def kernel(inputs, label, weight):
    lab3 = label.reshape(_NB, 1, _BB)
    loss, qst, perp, enc = pl.pallas_call(
        _vq_body,
        grid=(_NB,),
        in_specs=[
            pl.BlockSpec((_BB, EMBEDDING_DIM), lambda i: (i, 0)),
            pl.BlockSpec((1, 1, _BB), lambda i: (i, 0, 0)),
            pl.BlockSpec((NUM_EMBEDDINGS, EMBEDDING_DIM), lambda i: (0, 0)),
        ],
        out_specs=[
            pl.BlockSpec((1, 1), lambda i: (0, 0)),
            pl.BlockSpec((_BB, EMBEDDING_DIM), lambda i: (i, 0)),
            pl.BlockSpec((1, 1), lambda i: (0, 0)),
            pl.BlockSpec((_BB, NUM_EMBEDDINGS), lambda i: (i, 0)),
        ],
        out_shape=[
            jax.ShapeDtypeStruct((1, 1), jnp.float32),
            jax.ShapeDtypeStruct((BATCH, EMBEDDING_DIM), jnp.float32),
            jax.ShapeDtypeStruct((1, 1), jnp.float32),
            jax.ShapeDtypeStruct((BATCH, NUM_EMBEDDINGS), jnp.float32),
        ],
        scratch_shapes=[
            pltpu.SMEM((2,), jnp.float32),
            pltpu.VMEM((NUM_EMBEDDINGS,), jnp.float32),
        ],
    )(inputs, lab3, weight)
    return (loss[0, 0], qst, perp[0, 0], enc)


# speedup vs baseline: 5.2229x; 1.0866x over previous
import jax
import jax.numpy as jnp
from jax.experimental import pallas as pl
from jax.experimental.pallas import tpu as pltpu

NUM_EMBEDDINGS = 1024
EMBEDDING_DIM = 64
BATCH = 16384
COMMITMENT_COST = 0.25
DIVERGENCE_COST = 0.1

_BB = 512
_NB = BATCH // _BB


def _vq_body(x_ref, lab_ref, w_ref, loss_ref, qst_ref, perp_ref, enc_ref,
             sums_ref, counts_ref, colids_ref, wa_ref, iota_ref):
    i = pl.program_id(0)
    w = w_ref[...]

    @pl.when(i == 0)
    def _prep():
        colids_ref[...] = jax.lax.broadcasted_iota(
            jnp.int32, (_BB, NUM_EMBEDDINGS), 1)
        w2 = jnp.sum(w * w, axis=1, keepdims=True)
        wa_ref[...] = jnp.concatenate([-2.0 * w, w2], axis=1)
        iota_ref[...] = jax.lax.broadcasted_iota(
            jnp.int32, (NUM_EMBEDDINGS, 1), 0).astype(jnp.float32)

    x = x_ref[...]
    lab = lab_ref[0, 0, :]
    colids = colids_ref[...]

    xa = jnp.concatenate([x, jnp.ones((_BB, 1), jnp.float32)], axis=1)
    scores = jax.lax.dot_general(xa, wa_ref[...], (((1,), (1,)), ((), ())),
                                 preferred_element_type=jnp.float32)

    smin = jnp.min(scores, axis=1, keepdims=True)
    cenc = (scores <= smin).astype(jnp.float32)
    enc = (colids == lab[:, None]).astype(jnp.float32)

    close = jax.lax.dot_general(cenc, iota_ref[...], (((1,), (0,)), ((), ())),
                                preferred_element_type=jnp.float32)
    q = jax.lax.dot_general(enc, w, (((1,), (0,)), ((), ())),
                            preferred_element_type=jnp.float32)
    cnt = jax.lax.dot_general(jnp.ones((1, _BB), jnp.float32), enc,
                              (((1,), (0,)), ((), ())),
                              preferred_element_type=jnp.float32)

    ind = (lab.astype(jnp.float32) != close[:, 0]).astype(jnp.float32)
    x2 = jnp.sum(x * x, axis=1)
    dq = q - x
    s1 = jnp.sum(dq * dq)
    s2 = jnp.sum(ind * (smin[:, 0] + x2))

    enc_ref[...] = enc
    qst_ref[...] = x + dq

    @pl.when(i == 0)
    def _init():
        sums_ref[0] = s1
        sums_ref[1] = s2
        counts_ref[...] = cnt

    @pl.when(i > 0)
    def _acc():
        sums_ref[0] += s1
        sums_ref[1] += s2
        counts_ref[...] += cnt

    @pl.when(i == _NB - 1)
    def _finish():
        denom = float(BATCH * EMBEDDING_DIM)
        m1 = sums_ref[0] / denom
        m2 = sums_ref[1] / denom
        loss_ref[...] = jnp.reshape((1.0 + COMMITMENT_COST) * m1
                                    - (1.0 + DIVERGENCE_COST) * m2, (1, 1))
        p = counts_ref[...] / float(BATCH)
        perp_ref[...] = jnp.reshape(
            jnp.exp(-jnp.sum(p * jnp.log(p + 1e-10))), (1, 1))


def kernel(inputs, label, weight):
    lab3 = label.reshape(_NB, 1, _BB)
    loss, qst, perp, enc = pl.pallas_call(
        _vq_body,
        grid=(_NB,),
        in_specs=[
            pl.BlockSpec((_BB, EMBEDDING_DIM), lambda i: (i, 0)),
            pl.BlockSpec((1, 1, _BB), lambda i: (i, 0, 0)),
            pl.BlockSpec((NUM_EMBEDDINGS, EMBEDDING_DIM), lambda i: (0, 0)),
        ],
        out_specs=[
            pl.BlockSpec((1, 1), lambda i: (0, 0)),
            pl.BlockSpec((_BB, EMBEDDING_DIM), lambda i: (i, 0)),
            pl.BlockSpec((1, 1), lambda i: (0, 0)),
            pl.BlockSpec((_BB, NUM_EMBEDDINGS), lambda i: (i, 0)),
        ],
        out_shape=[
            jax.ShapeDtypeStruct((1, 1), jnp.float32),
            jax.ShapeDtypeStruct((BATCH, EMBEDDING_DIM), jnp.float32),
            jax.ShapeDtypeStruct((1, 1), jnp.float32),
            jax.ShapeDtypeStruct((BATCH, NUM_EMBEDDINGS), jnp.float32),
        ],
        scratch_shapes=[
            pltpu.SMEM((2,), jnp.float32),
            pltpu.VMEM((1, NUM_EMBEDDINGS), jnp.float32),
            pltpu.VMEM((_BB, NUM_EMBEDDINGS), jnp.int32),
            pltpu.VMEM((NUM_EMBEDDINGS, EMBEDDING_DIM + 1), jnp.float32),
            pltpu.VMEM((NUM_EMBEDDINGS, 1), jnp.float32),
        ],
    )(inputs, lab3, weight)
    return (loss[0, 0], qst, perp[0, 0], enc)


# speedup vs baseline: 5.6540x; 1.0825x over previous
import jax
import jax.numpy as jnp
from jax.experimental import pallas as pl
from jax.experimental.pallas import tpu as pltpu

NUM_EMBEDDINGS = 1024
EMBEDDING_DIM = 64
BATCH = 16384
COMMITMENT_COST = 0.25
DIVERGENCE_COST = 0.1

_BB = 512
_NB = BATCH // _BB


def _vq_body(x_ref, lab_ref, w_ref, loss_ref, qst_ref, perp_ref, enc_ref,
             sums_ref, counts_ref, colids_ref, wa_ref, iota_ref):
    i = pl.program_id(0)
    w = w_ref[...]

    @pl.when(i == 0)
    def _prep():
        colids_ref[...] = jax.lax.broadcasted_iota(
            jnp.int32, (_BB, NUM_EMBEDDINGS), 1)
        w2 = jnp.sum(w * w, axis=1, keepdims=True)
        wa_ref[...] = jnp.concatenate([-2.0 * w, w2], axis=1)
        iota_ref[...] = jnp.concatenate(
            [w, jax.lax.broadcasted_iota(
                jnp.int32, (NUM_EMBEDDINGS, 1), 0).astype(jnp.float32)],
            axis=1)

    x = x_ref[...]
    lab = lab_ref[0, 0, :]
    colids = colids_ref[...]

    xa = jnp.concatenate([x, jnp.ones((_BB, 1), jnp.float32)], axis=1)
    scores = jax.lax.dot_general(xa, wa_ref[...], (((1,), (1,)), ((), ())),
                                 preferred_element_type=jnp.float32)

    smin = jnp.min(scores, axis=1, keepdims=True)
    cenc = (scores <= smin).astype(jnp.float32)
    enc = (colids == lab[:, None]).astype(jnp.float32)

    cqc = jax.lax.dot_general(cenc, iota_ref[...], (((1,), (0,)), ((), ())),
                              preferred_element_type=jnp.float32)
    q = jax.lax.dot_general(enc, w, (((1,), (0,)), ((), ())),
                            preferred_element_type=jnp.float32)
    cnt = jax.lax.dot_general(jnp.ones((1, _BB), jnp.float32), enc,
                              (((1,), (0,)), ((), ())),
                              preferred_element_type=jnp.float32)

    cq = cqc[:, :EMBEDDING_DIM]
    close = cqc[:, EMBEDDING_DIM]
    ind = (lab.astype(jnp.float32) != close).astype(jnp.float32)
    dq = q - x
    dcq = cq - x
    s1 = jnp.sum(dq * dq)
    s2 = jnp.sum(ind[:, None] * (dcq * dcq))

    enc_ref[...] = enc
    qst_ref[...] = x + dq

    @pl.when(i == 0)
    def _init():
        sums_ref[0] = s1
        sums_ref[1] = s2
        counts_ref[...] = cnt

    @pl.when(i > 0)
    def _acc():
        sums_ref[0] += s1
        sums_ref[1] += s2
        counts_ref[...] += cnt

    @pl.when(i == _NB - 1)
    def _finish():
        denom = float(BATCH * EMBEDDING_DIM)
        m1 = sums_ref[0] / denom
        m2 = sums_ref[1] / denom
        loss_ref[...] = jnp.reshape((1.0 + COMMITMENT_COST) * m1
                                    - (1.0 + DIVERGENCE_COST) * m2, (1, 1))
        p = counts_ref[...] / float(BATCH)
        perp_ref[...] = jnp.reshape(
            jnp.exp(-jnp.sum(p * jnp.log(p + 1e-10))), (1, 1))


def kernel(inputs, label, weight):
    lab3 = label.reshape(_NB, 1, _BB)
    loss, qst, perp, enc = pl.pallas_call(
        _vq_body,
        grid=(_NB,),
        in_specs=[
            pl.BlockSpec((_BB, EMBEDDING_DIM), lambda i: (i, 0)),
            pl.BlockSpec((1, 1, _BB), lambda i: (i, 0, 0)),
            pl.BlockSpec((NUM_EMBEDDINGS, EMBEDDING_DIM), lambda i: (0, 0)),
        ],
        out_specs=[
            pl.BlockSpec((1, 1), lambda i: (0, 0)),
            pl.BlockSpec((_BB, EMBEDDING_DIM), lambda i: (i, 0)),
            pl.BlockSpec((1, 1), lambda i: (0, 0)),
            pl.BlockSpec((_BB, NUM_EMBEDDINGS), lambda i: (i, 0)),
        ],
        out_shape=[
            jax.ShapeDtypeStruct((1, 1), jnp.float32),
            jax.ShapeDtypeStruct((BATCH, EMBEDDING_DIM), jnp.float32),
            jax.ShapeDtypeStruct((1, 1), jnp.float32),
            jax.ShapeDtypeStruct((BATCH, NUM_EMBEDDINGS), jnp.float32),
        ],
        scratch_shapes=[
            pltpu.SMEM((2,), jnp.float32),
            pltpu.VMEM((1, NUM_EMBEDDINGS), jnp.float32),
            pltpu.VMEM((_BB, NUM_EMBEDDINGS), jnp.int32),
            pltpu.VMEM((NUM_EMBEDDINGS, EMBEDDING_DIM + 1), jnp.float32),
            pltpu.VMEM((NUM_EMBEDDINGS, EMBEDDING_DIM + 1), jnp.float32),
        ],
    )(inputs, lab3, weight)
    return (loss[0, 0], qst, perp[0, 0], enc)
